# Initial kernel scaffold; baseline (speedup 1.0000x reference)
import jax
import jax.numpy as jnp
from jax.experimental import pallas as pl


def kernel(input_data):
    raise NotImplementedError("write your pallas kernel here")


# speedup vs baseline: 124.4278x; 124.4278x over previous
import functools

import numpy as np
import jax
import jax.numpy as jnp
from jax import lax
from jax.experimental import pallas as pl
from jax.experimental.pallas import tpu as pltpu
from jax.experimental.pallas import tpu_sc as plsc

_IMAGE = 64
_PSIZE = 8
_STRIDE = 4
_NP = 15
_BATCH = 4
_CHANNELS = 64
_BC = _BATCH * _CHANNELS
_K = _PSIZE * _PSIZE
_NPATCH = _NP * _NP
_XLEN = _NPATCH * _K
_PAD = 16
_XLEN_PAD = _XLEN + _PAD
_NPIX = _IMAGE * _IMAGE
_LANES = 16


def _build_tables():
    idx = np.full((4, _NPIX), _XLEN, dtype=np.int32)
    cnt = np.zeros((_NPIX,), dtype=np.float32)
    for t, (da, db) in enumerate([(0, 0), (0, 1), (1, 0), (1, 1)]):
        for y in range(_IMAGE):
            a = y // _STRIDE - da
            i = y % _STRIDE + _STRIDE * da
            if not 0 <= a < _NP:
                continue
            for x in range(_IMAGE):
                b = x // _STRIDE - db
                j = x % _STRIDE + _STRIDE * db
                if not 0 <= b < _NP:
                    continue
                idx[t, y * _IMAGE + x] = (a * _NP + b) * _K + i * _PSIZE + j
                cnt[y * _IMAGE + x] += 1.0
    return idx, (1.0 / cnt).astype(np.float32)


_IDX_TAB, _RECIP_TAB = _build_tables()


def _sc_core_counts():
    try:
        info = plsc.get_sparse_core_info()
        return info.num_cores, info.num_subcores
    except Exception:
        return 2, 16


@functools.cache
def _make_sc_kernel():
    nc, ns = _sc_core_counts()
    nw = nc * ns
    rows_per = _BC // nw
    mesh = plsc.VectorSubcoreMesh(core_axis_name="c", subcore_axis_name="s")

    @functools.partial(
        pl.kernel,
        mesh=mesh,
        out_type=jax.ShapeDtypeStruct((_BC, _NPIX), jnp.float32),
        compiler_params=pltpu.CompilerParams(
            needs_layout_passes=False, use_tc_tiling_on_sc=False
        ),
        scratch_types=[
            pltpu.VMEM((_XLEN_PAD,), jnp.float32),
            pltpu.VMEM((_NPIX,), jnp.float32),
            pltpu.VMEM((4, _NPIX), jnp.int32),
            pltpu.VMEM((_NPIX,), jnp.float32),
        ],
    )
    def k(x_hbm, idx_hbm, recip_hbm, out_hbm, xbuf, obuf, ibuf, rbuf):
        wid = lax.axis_index("s") * nc + lax.axis_index("c")
        pltpu.sync_copy(idx_hbm, ibuf)
        pltpu.sync_copy(recip_hbm, rbuf)
        xbuf[pl.ds(_XLEN, _PAD)] = jnp.zeros((_PAD,), jnp.float32)

        def body(v, _):
            sl = pl.ds(v * _LANES, _LANES)
            acc = plsc.load_gather(xbuf, [ibuf[0, sl]])
            acc = acc + plsc.load_gather(xbuf, [ibuf[1, sl]])
            acc = acc + plsc.load_gather(xbuf, [ibuf[2, sl]])
            acc = acc + plsc.load_gather(xbuf, [ibuf[3, sl]])
            obuf[sl] = acc * rbuf[sl]
            return 0

        for row in range(rows_per):
            bc = wid * rows_per + row
            pltpu.sync_copy(x_hbm.at[bc], xbuf.at[pl.ds(0, _XLEN)])
            lax.fori_loop(0, _NPIX // _LANES, body, 0, unroll=4)
            pltpu.sync_copy(obuf, out_hbm.at[bc])

    return k


def kernel(input_data):
    x2 = input_data.reshape(_BC, _XLEN)
    out = _make_sc_kernel()(x2, jnp.asarray(_IDX_TAB), jnp.asarray(_RECIP_TAB))
    return out.reshape(_BATCH, _CHANNELS, _IMAGE, _IMAGE)
